# baseline (speedup 1.0000x reference)
import functools

import jax
import jax.numpy as jnp
from jax import lax
from jax.experimental import pallas as pl
from jax.experimental.pallas import tpu as pltpu
from jax.experimental.pallas import tpu_sc as plsc

_E = 800000
_C_IN = 16
_C_HID = 64
_NW = 32
_CHUNK = 800
_GSLICE = 128
_NR = _CHUNK // _GSLICE
_NSTEP = 32
_PER_W = _CHUNK * _NSTEP
_E_PAD = _PER_W * _NW


def _gather_body(nf_hbm, src_hbm, dst_hbm, out_hbm,
                 sidx0, sidx1, didx0, didx1, srows0, srows1, drows0, drows1,
                 tbuf0, tbuf1, shared_nf, si0, si1, sg0, sg1, sw0, sw1,
                 *, h, nstep, e_span):
    SIDX, DIDX = [sidx0, sidx1], [didx0, didx1]
    SROWS, DROWS = [srows0, srows1], [drows0, drows1]
    TBUF, SI, SG, SW = [tbuf0, tbuf1], [si0, si1], [sg0, sg1], [sw0, sw1]

    per_w = _CHUNK * nstep
    wid = lax.axis_index("s") * 2 + lax.axis_index("c")
    base = wid * per_w
    feat16 = jnp.arange(_C_IN, dtype=jnp.int32)
    half = nstep // 2

    def issue_idx(c, b):
        off = h * e_span + base + c * _CHUNK
        pltpu.async_copy(src_hbm.at[pl.ds(off, _CHUNK)], SIDX[b], SI[b])
        pltpu.async_copy(dst_hbm.at[pl.ds(off, _CHUNK)], DIDX[b], SI[b])

    def wait_idx(b):
        pltpu.make_async_copy(src_hbm.at[pl.ds(0, _CHUNK)], SIDX[b], SI[b]).wait()
        pltpu.make_async_copy(dst_hbm.at[pl.ds(0, _CHUNK)], DIDX[b], SI[b]).wait()

    def issue_gather(b):
        pltpu.async_copy(shared_nf.at[SIDX[b]], SROWS[b], SG[b])
        pltpu.async_copy(shared_nf.at[DIDX[b]], DROWS[b], SG[b])

    def wait_gather(b):
        pltpu.make_async_copy(nf_hbm.at[pl.ds(0, _CHUNK)], SROWS[b], SG[b]).wait()
        pltpu.make_async_copy(nf_hbm.at[pl.ds(0, _CHUNK)], DROWS[b], SG[b]).wait()

    def compute(b):
        @plsc.parallel_loop(0, _CHUNK, 1, unroll=8)
        def _(i):
            s = SROWS[b][i] + DROWS[b][i]
            plsc.store_scatter(TBUF[b], [feat16, jnp.full((_C_IN,), i, jnp.int32)], s)

    def issue_write(c, b):
        off = base + c * _CHUNK
        pltpu.async_copy(TBUF[b],
                         out_hbm.at[:, pl.ds(off, _CHUNK)], SW[b])

    def wait_write(b):
        pltpu.make_async_copy(TBUF[b],
                              out_hbm.at[:, pl.ds(0, _CHUNK)], SW[b]).wait()

    @pl.when(lax.axis_index("s") == 0)
    def _():
        pltpu.sync_copy(nf_hbm, shared_nf)
    plsc.subcore_barrier()

    issue_idx(0, 0)
    wait_idx(0)
    issue_gather(0)
    issue_idx(1, 1)

    def iter_g(g, carry):
        for b in (0, 1):
            c = 2 * g + b
            nb = 1 - b

            def stage_next():
                wait_idx(nb)
                issue_gather(nb)
            if b == 0:
                stage_next()
            else:
                pl.when(g < half - 1)(stage_next)

            wait_gather(b)
            pl.when(g >= 1)(lambda: wait_write(b))
            compute(b)
            issue_write(c, b)
            pl.when(g < half - 1)(lambda: issue_idx(c + 2, b))
        return carry

    lax.fori_loop(0, half, iter_g, 0)
    wait_write(0)
    wait_write(1)


def _sc_gather(node_features, src1d, dst1d, h, nsplit):
    e_span = _E_PAD // nsplit
    nstep = _NSTEP // nsplit
    mesh = plsc.VectorSubcoreMesh(core_axis_name="c", subcore_axis_name="s")
    f = pl.kernel(
        functools.partial(_gather_body, h=h, nstep=nstep, e_span=e_span),
        out_type=jax.ShapeDtypeStruct((_C_IN, e_span), jnp.float32),
        mesh=mesh,
        compiler_params=pltpu.CompilerParams(use_tc_tiling_on_sc=False,
                                             needs_layout_passes=False),
        scratch_types=(
            [pltpu.VMEM((_CHUNK,), jnp.int32) for _ in range(4)]
            + [pltpu.VMEM((_CHUNK, _C_IN), jnp.float32) for _ in range(4)]
            + [pltpu.VMEM((_C_IN, _CHUNK), jnp.float32) for _ in range(2)]
            + [pltpu.VMEM_SHARED((50000, _C_IN), jnp.float32)]
            + [pltpu.SemaphoreType.DMA for _ in range(6)]
        ),
    )
    return f(node_features, src1d, dst1d)


def _dense_body(sum3, eft, w1, b1, w2, b2, out_ref):
    ns = sum3[...].reshape(_C_IN, -1)
    x = jnp.concatenate([ns, eft[...]], axis=0)
    h = jnp.maximum(
        jnp.dot(w1[...], x, preferred_element_type=jnp.float32) + b1[...], 0.0)
    t = jnp.dot(w2[...], h, preferred_element_type=jnp.float32) + b2[...]
    mean = jnp.mean(t, axis=0, keepdims=True)
    var = jnp.mean((t - mean) ** 2, axis=0, keepdims=True)
    tn = (t - mean) * lax.rsqrt(var + 1e-5)
    out_ref[...] = eft[...] + jnp.maximum(tn, 0.0)


def _dense_half(sum3, eft, w1, b1, w2, b2, prev, block0, nblk, block_e=4096):
    body = _dense_body
    in_specs = [
        pl.BlockSpec((_C_IN, block_e // 128, 128), lambda i: (0, i, 0)),
        pl.BlockSpec((_C_IN, block_e), lambda i: (0, i + block0)),
        pl.BlockSpec((2 * _C_HID, 2 * _C_IN), lambda i: (0, 0)),
        pl.BlockSpec((2 * _C_HID, 1), lambda i: (0, 0)),
        pl.BlockSpec((_C_IN, 2 * _C_HID), lambda i: (0, 0)),
        pl.BlockSpec((_C_IN, 1), lambda i: (0, 0)),
    ]
    args = [sum3, eft, w1, b1, w2, b2]
    kwargs = {}
    if prev is not None:
        def body(sum3, eft, w1, b1, w2, b2, prev_ref, out_ref):
            _dense_body(sum3, eft, w1, b1, w2, b2, out_ref)
        in_specs.append(pl.BlockSpec(memory_space=pl.ANY))
        args.append(prev)
        kwargs["input_output_aliases"] = {6: 0}
    return pl.pallas_call(
        body,
        grid=(nblk,),
        in_specs=in_specs,
        out_specs=pl.BlockSpec((_C_IN, block_e), lambda i: (0, i + block0)),
        out_shape=jax.ShapeDtypeStruct((_C_IN, _E), jnp.float32),
        **kwargs,
    )(*args)


def kernel(node_features, edge_index, edge_features,
           W1a, b1a, W2a, b2a, W1b, b1b, W2b, b2b):
    pad = _E_PAD - _E
    src1d = jnp.concatenate([edge_index[0], jnp.zeros((pad,), jnp.int32)])
    dst1d = jnp.concatenate([edge_index[1], jnp.zeros((pad,), jnp.int32)])

    nsplit = 4
    e_span = _E_PAD // nsplit
    sums = [_sc_gather(node_features, src1d, dst1d, h, nsplit)
            .reshape(_C_IN, e_span // 128, 128) for h in range(nsplit)]

    w1 = jnp.zeros((2 * _C_HID, 2 * _C_IN), jnp.float32)
    w1 = w1.at[0:_C_HID, 0:_C_IN].set(W1a)
    w1 = w1.at[_C_HID:, _C_IN:].set(W1b)
    b1 = jnp.concatenate([b1a, b1b]).reshape(2 * _C_HID, 1)
    w2 = jnp.concatenate([W2a, W2b], axis=1)
    b2 = (b2a + b2b).reshape(_C_IN, 1)

    eft = edge_features.T
    block_e = 4096
    blocks_per_half = e_span // block_e
    total_blocks = pl.cdiv(_E, block_e)
    out = None
    for h in range(nsplit):
        block0 = h * blocks_per_half
        nblk = min(blocks_per_half, total_blocks - block0)
        out = _dense_half(sums[h], eft, w1, b1, w2, b2, out,
                          block0, nblk, block_e)
    return out.T
